# baseline (speedup 1.0000x reference)
import math

import jax
import jax.numpy as jnp
from jax.experimental import pallas as pl
from jax.experimental.pallas import tpu as pltpu


def _ffn_kernel(x_ref, w27_ref, bias_ref, w2_ref, b2_ref, o_ref, *, H, W):
    N = H * W
    C = x_ref.shape[-1]
    Hh = H // 2

    x3 = x_ref[0].astype(jnp.bfloat16).reshape(H, W, C)
    zc = jnp.zeros((H, 1, C), x3.dtype)
    xm = jnp.concatenate([zc, x3[:, :W - 1]], axis=1)
    xp = jnp.concatenate([x3[:, 1:], zc], axis=1)
    x9 = jnp.concatenate([xm, x3, xp], axis=2)

    zr = jnp.zeros((1, W, 3 * C), x3.dtype)
    xdn = jnp.concatenate([zr, x9[:H - 1]], axis=0)
    xup = jnp.concatenate([x9[1:], zr], axis=0)

    inv_sqrt2 = jnp.float32(0.7071067811865476)
    for lo, hi in ((0, Hh), (Hh, H)):
        x27 = jnp.concatenate(
            [xdn[lo:hi], x9[lo:hi], xup[lo:hi]],
            axis=2).reshape((hi - lo) * W, 9 * C)

        conv = jnp.dot(x27, w27_ref[...], preferred_element_type=jnp.float32)
        conv = conv + bias_ref[lo * W:hi * W]

        g = 0.5 * conv * (1.0 + jax.lax.erf(conv * inv_sqrt2))

        out = jnp.dot(g.astype(w2_ref.dtype), w2_ref[...],
                      preferred_element_type=jnp.float32)
        o_ref[0, lo * W:hi * W] = (out + b2_ref[...]).astype(o_ref.dtype)


def _fused_ffn(x, w27, bias_field, w2, b2, *, H, W, interpret=False):
    B, N, C = x.shape
    hid = w2.shape[0]
    assert N == H * W

    def body(*refs):
        _ffn_kernel(*refs, H=H, W=W)

    return pl.pallas_call(
        body,
        out_shape=jax.ShapeDtypeStruct((B, N, C), jnp.float32),
        grid_spec=pltpu.PrefetchScalarGridSpec(
            num_scalar_prefetch=0,
            grid=(B,),
            in_specs=[
                pl.BlockSpec((1, N, C), lambda b: (b, 0, 0)),
                pl.BlockSpec((9 * C, hid), lambda b: (0, 0)),
                pl.BlockSpec((N, hid), lambda b: (0, 0)),
                pl.BlockSpec((hid, C), lambda b: (0, 0)),
                pl.BlockSpec((1, C), lambda b: (0, 0)),
            ],
            out_specs=pl.BlockSpec((1, N, C), lambda b: (b, 0, 0)),
        ),
        compiler_params=pltpu.CompilerParams(
            dimension_semantics=("parallel",),
            vmem_limit_bytes=100 * 1024 * 1024,
        ),
        cost_estimate=pl.CostEstimate(
            flops=2 * B * N * 9 * C * hid + 2 * B * N * hid * C,
            transcendentals=B * N * hid,
            bytes_accessed=(B * N * C * 4 + B * N * C * 4 + N * hid * 4
                            + (9 * C * hid + hid * C) * 2),
        ),
        interpret=interpret,
    )(x, w27, bias_field, w2, b2.reshape(1, C).astype(jnp.float32))


def _prep_weights(w1, b1, dw_w, dw_b, H, W):
    C, hid = w1.shape
    w27 = (w1[None, None] * dw_w[:, :, None, :]).astype(jnp.bfloat16)
    w27 = w27.reshape(9 * C, hid)

    ksum = dw_w.sum((0, 1))
    row0, row2 = dw_w[0].sum(0), dw_w[2].sum(0)
    col0, col2 = dw_w[:, 0].sum(0), dw_w[:, 2].sum(0)
    eh = jnp.zeros((H, 1, 1), jnp.float32)
    top = eh.at[0].set(1.0)
    bot = eh.at[H - 1].set(1.0)
    ew = jnp.zeros((1, W, 1), jnp.float32)
    lef = ew.at[:, 0].set(1.0)
    rig = ew.at[:, W - 1].set(1.0)
    miss = (top * row0 + bot * row2 + lef * col0 + rig * col2
            - top * lef * dw_w[0, 0] - top * rig * dw_w[0, 2]
            - bot * lef * dw_w[2, 0] - bot * rig * dw_w[2, 2])
    bias_field = dw_b + b1 * (ksum - miss)
    return w27, bias_field.reshape(H * W, hid)


def kernel(x, w1, b1, w2, b2, dw_w, dw_b):
    B, N, C = x.shape
    H = W = math.isqrt(N)
    w27, bias_field = _prep_weights(w1, b1, dw_w, dw_b, H, W)
    return _fused_ffn(x, w27, bias_field, w2.astype(jnp.bfloat16), b2,
                      H=H, W=W)


# speedup vs baseline: 1.3632x; 1.3632x over previous
import math

import jax
import jax.numpy as jnp
from jax.experimental import pallas as pl
from jax.experimental.pallas import tpu as pltpu

_KPAD = 16


def _ffn_kernel(x_ref, w27_ref, mask_ref, w2_ref, b2_ref, o_ref, x27,
                *, H, W):
    N = H * W
    C = x_ref.shape[-1]

    x2 = x_ref[0].astype(jnp.bfloat16)
    x3 = x2.reshape(H, W, C)
    zc = jnp.zeros((H, 1, C), x2.dtype)
    xm2 = jnp.concatenate([zc, x3[:, :W - 1]], axis=1).reshape(N, C)
    xp2 = jnp.concatenate([x3[:, 1:], zc], axis=1).reshape(N, C)
    zw = jnp.zeros((W, C), x2.dtype)

    for dh in range(3):
        for dw, src in enumerate((xm2, x2, xp2)):
            col = (dh * 3 + dw) * C
            if dh == 0:
                x27[W:, col:col + C] = src[:N - W]
                x27[:W, col:col + C] = zw
            elif dh == 1:
                x27[:, col:col + C] = src
            else:
                x27[:N - W, col:col + C] = src[W:]
                x27[N - W:, col:col + C] = zw
    x27[:, 9 * C:] = mask_ref[...]

    t = jnp.dot(x27[...], w27_ref[...], preferred_element_type=jnp.float32)

    g = t * (1.0 + jax.lax.erf(t))

    out = jnp.dot(g.astype(w2_ref.dtype), w2_ref[...],
                  preferred_element_type=jnp.float32)
    o_ref[0] = (out + b2_ref[...]).astype(o_ref.dtype)


def _fused_ffn(x, w27, masks, w2s, b2, *, H, W, interpret=False):
    B, N, C = x.shape
    hid = w2s.shape[0]
    assert N == H * W
    kdim = 9 * C + _KPAD

    def body(*refs):
        _ffn_kernel(*refs, H=H, W=W)

    return pl.pallas_call(
        body,
        out_shape=jax.ShapeDtypeStruct((B, N, C), jnp.float32),
        grid_spec=pltpu.PrefetchScalarGridSpec(
            num_scalar_prefetch=0,
            grid=(B,),
            in_specs=[
                pl.BlockSpec((1, N, C), lambda b: (b, 0, 0)),
                pl.BlockSpec((kdim, hid), lambda b: (0, 0)),
                pl.BlockSpec((N, _KPAD), lambda b: (0, 0)),
                pl.BlockSpec((hid, C), lambda b: (0, 0)),
                pl.BlockSpec((1, C), lambda b: (0, 0)),
            ],
            out_specs=pl.BlockSpec((1, N, C), lambda b: (b, 0, 0)),
            scratch_shapes=[pltpu.VMEM((N, kdim), jnp.bfloat16)],
        ),
        compiler_params=pltpu.CompilerParams(
            dimension_semantics=("parallel",),
            vmem_limit_bytes=100 * 1024 * 1024,
        ),
        cost_estimate=pl.CostEstimate(
            flops=2 * B * N * kdim * hid + 2 * B * N * hid * C,
            transcendentals=B * N * hid,
            bytes_accessed=(B * N * C * 4 + B * N * C * 4
                            + (kdim * hid + hid * C) * 2 + N * _KPAD * 2),
        ),
        interpret=interpret,
    )(x, w27, masks, w2s, b2.reshape(1, C).astype(jnp.float32))


def _prep_weights(w1, b1, w2, dw_w, dw_b, H, W):
    C, hid = w1.shape
    inv_sqrt2 = 0.7071067811865476

    taps = (w1[None, None] * dw_w[:, :, None, :]).reshape(9 * C, hid)

    ksum = dw_w.sum((0, 1))
    row0, row2 = dw_w[0].sum(0), dw_w[2].sum(0)
    col0, col2 = dw_w[:, 0].sum(0), dw_w[:, 2].sum(0)
    vals = jnp.stack([
        dw_b + b1 * ksum,
        -b1 * row0,
        -b1 * row2,
        -b1 * col0,
        -b1 * col2,
        b1 * dw_w[0, 0],
        b1 * dw_w[0, 2],
        b1 * dw_w[2, 0],
        b1 * dw_w[2, 2],
    ])
    vals = jnp.concatenate(
        [vals, jnp.zeros((_KPAD - 9, hid), jnp.float32)], axis=0)
    w27 = (jnp.concatenate([taps, vals], axis=0)
           * inv_sqrt2).astype(jnp.bfloat16)

    hh = jnp.zeros((H, 1), jnp.float32)
    top, bot = hh.at[0].set(1.0), hh.at[H - 1].set(1.0)
    ww = jnp.zeros((1, W), jnp.float32)
    lef, rig = ww.at[:, 0].set(1.0), ww.at[:, W - 1].set(1.0)
    one = jnp.ones((H, W), jnp.float32)
    masks = jnp.stack([
        one, top * one, bot * one, lef * one, rig * one,
        top * lef, top * rig, bot * lef, bot * rig,
    ], axis=-1).reshape(H * W, 9)
    masks = jnp.concatenate(
        [masks, jnp.zeros((H * W, _KPAD - 9), jnp.float32)],
        axis=1).astype(jnp.bfloat16)

    w2s = (w2 * (0.5 * math.sqrt(2.0))).astype(jnp.bfloat16)
    return w27, masks, w2s


def kernel(x, w1, b1, w2, b2, dw_w, dw_b):
    B, N, C = x.shape
    H = W = math.isqrt(N)
    w27, masks, w2s = _prep_weights(w1, b1, w2, dw_w, dw_b, H, W)
    return _fused_ffn(x, w27, masks, w2s, b2, H=H, W=W)


# speedup vs baseline: 1.4617x; 1.0723x over previous
import math

import jax
import jax.numpy as jnp
from jax.experimental import pallas as pl
from jax.experimental.pallas import tpu as pltpu


def _ffn_kernel(x_ref, w27_ref, bias_ref, w2_ref, b2_ref, o_ref, *, H, W):
    N = H * W
    C = x_ref.shape[-1]

    x3 = x_ref[0].astype(jnp.bfloat16).reshape(H, W, C)
    zc = jnp.zeros((H, 1, C), x3.dtype)
    xm = jnp.concatenate([zc, x3[:, :W - 1]], axis=1)
    xp = jnp.concatenate([x3[:, 1:], zc], axis=1)
    x9 = jnp.concatenate([xm, x3, xp], axis=2)

    zr = jnp.zeros((1, W, 3 * C), x3.dtype)
    xdn = jnp.concatenate([zr, x9[:H - 1]], axis=0)
    xup = jnp.concatenate([x9[1:], zr], axis=0)
    x27 = jnp.concatenate([xdn, x9, xup], axis=2).reshape(N, 9 * C)

    t = jnp.dot(x27, w27_ref[...], preferred_element_type=jnp.float32)
    t = t + bias_ref[...]

    g = t * (1.0 + jax.lax.erf(t))

    out = jnp.dot(g.astype(w2_ref.dtype), w2_ref[...],
                  preferred_element_type=jnp.float32)
    o_ref[0] = (out + b2_ref[...]).astype(o_ref.dtype)


def _fused_ffn(x, w27, bias_field, w2s, b2, *, H, W, interpret=False):
    B, N, C = x.shape
    hid = w2s.shape[0]
    assert N == H * W

    def body(*refs):
        _ffn_kernel(*refs, H=H, W=W)

    return pl.pallas_call(
        body,
        out_shape=jax.ShapeDtypeStruct((B, N, C), jnp.float32),
        grid_spec=pltpu.PrefetchScalarGridSpec(
            num_scalar_prefetch=0,
            grid=(B,),
            in_specs=[
                pl.BlockSpec((1, N, C), lambda b: (b, 0, 0)),
                pl.BlockSpec((9 * C, hid), lambda b: (0, 0)),
                pl.BlockSpec((N, hid), lambda b: (0, 0)),
                pl.BlockSpec((hid, C), lambda b: (0, 0)),
                pl.BlockSpec((1, C), lambda b: (0, 0)),
            ],
            out_specs=pl.BlockSpec((1, N, C), lambda b: (b, 0, 0)),
        ),
        compiler_params=pltpu.CompilerParams(
            dimension_semantics=("parallel",),
            vmem_limit_bytes=100 * 1024 * 1024,
        ),
        cost_estimate=pl.CostEstimate(
            flops=2 * B * N * 9 * C * hid + 2 * B * N * hid * C,
            transcendentals=B * N * hid,
            bytes_accessed=(B * N * C * 4 + B * N * C * 4 + N * hid * 4
                            + (9 * C * hid + hid * C) * 2),
        ),
        interpret=interpret,
    )(x, w27, bias_field, w2s, b2.reshape(1, C).astype(jnp.float32))


def _prep_weights(w1, b1, w2, dw_w, dw_b, H, W):
    C, hid = w1.shape
    inv_sqrt2 = 0.7071067811865476

    w27 = (w1[None, None] * dw_w[:, :, None, :]) * inv_sqrt2
    w27 = w27.reshape(9 * C, hid).astype(jnp.bfloat16)

    ksum = dw_w.sum((0, 1))
    row0, row2 = dw_w[0].sum(0), dw_w[2].sum(0)
    col0, col2 = dw_w[:, 0].sum(0), dw_w[:, 2].sum(0)
    eh = jnp.zeros((H, 1, 1), jnp.float32)
    top = eh.at[0].set(1.0)
    bot = eh.at[H - 1].set(1.0)
    ew = jnp.zeros((1, W, 1), jnp.float32)
    lef = ew.at[:, 0].set(1.0)
    rig = ew.at[:, W - 1].set(1.0)
    miss = (top * row0 + bot * row2 + lef * col0 + rig * col2
            - top * lef * dw_w[0, 0] - top * rig * dw_w[0, 2]
            - bot * lef * dw_w[2, 0] - bot * rig * dw_w[2, 2])
    bias_field = (dw_b + b1 * (ksum - miss)) * inv_sqrt2

    w2s = (w2 * (0.5 * math.sqrt(2.0))).astype(jnp.bfloat16)
    return w27, bias_field.reshape(H * W, hid), w2s


def kernel(x, w1, b1, w2, b2, dw_w, dw_b):
    B, N, C = x.shape
    H = W = math.isqrt(N)
    w27, bias_field, w2s = _prep_weights(w1, b1, w2, dw_w, dw_b, H, W)
    return _fused_ffn(x, w27, bias_field, w2s, b2, H=H, W=W)
